# baseline (speedup 1.0000x reference)
import functools

import jax
import jax.numpy as jnp
from jax import lax
from jax.experimental import pallas as pl
from jax.experimental.pallas import tpu as pltpu
from jax.experimental.pallas import tpu_sc as plsc

N = 10000
E = 160000
C = 64
NC, NS = 2, 16
ET = E // NS
B = 80
NCH = ET // B
NZ = N // NS
NZC = 125

_RELS = [("cm", "circRNA", "miRNA"), ("md", "miRNA", "disease"),
         ("cd", "circRNA", "disease"), ("mc", "miRNA", "circRNA"),
         ("dm", "disease", "miRNA"), ("dc", "disease", "circRNA")]
_GROUPS = {"circRNA": ("mc", "dc"), "miRNA": ("cm", "dm"), "disease": ("md", "cd")}


def _edge_body(H, F,
               src1, dst1, ab1, h1,
               src2, dst2, ab2, h2,
               out, w1, w2,
               srcb, dst2d, w8, den8, alpha8, asrows, adrows, rows, zbuf8,
               sem1, sem2, sem3, ab1_s, ab2_s, den_s, out_s):
    FH = F // NC
    c = lax.axis_index("c")
    s = lax.axis_index("s")
    lane = lax.broadcasted_iota(jnp.int32, (16,), 0)
    lane8 = jnp.bitwise_and(lane, 7)
    lane4p = jnp.bitwise_and(lane, 3) + 4
    mh = lane < H
    z16 = jnp.zeros((16,), jnp.float32)
    base = s * NZ

    def i16(v):
        return jnp.full((16,), v, jnp.int32)

    def zw(i, carry):
        plsc.store_scatter(w8, [i16(i), lane8], z16)
        return carry

    def zr(i, carry):
        for jb in range(FH // 16):
            rows[i, pl.ds(jb * 16, 16)] = z16
        return carry

    def zb(i, carry):
        plsc.store_scatter(zbuf8, [i16(i), lane8], z16)
        return carry

    lax.fori_loop(0, NZC, zb, 0)

    def pass_a(src_hbm, dst3d_hbm, ab_s, off, w_hbm):
        lax.fori_loop(0, B, zw, 0)
        pltpu.sync_copy(dst3d_hbm.at[s], dst2d)

        def chunk(j, carry):
            pltpu.sync_copy(src_hbm.at[s, j], srcb)
            d1 = pltpu.async_copy(ab_s.at[srcb], asrows, sem1)
            d2 = pltpu.async_copy(ab_s.at[dst2d.at[j]], adrows, sem2)
            d1.wait()
            d2.wait()

            def edge(e2, carry2):
                erow = i16(0) + e2
                av = plsc.load_gather(asrows, [erow, lane8])
                dv = plsc.load_gather(adrows, [erow, lane4p])
                x = av + dv
                e = jnp.maximum(x, 0.2 * x)
                w = jnp.exp(e)
                plsc.store_scatter(w8, [erow, lane + off], w, mask=mh)
                return carry2

            lax.fori_loop(0, B, edge, 0, unroll=2)
            pltpu.sync_copy(w8, den_s.at[dst2d.at[j]], add=True)
            pltpu.sync_copy(w8, w_hbm.at[c, s, j])
            return carry

        lax.fori_loop(0, NCH, chunk, 0)

    head_base = (c * FH) // C
    nh = max(1, FH // C)
    cols = [i16(0) + (head_base + hh) for hh in range(nh)]

    def pass_b(src_hbm, dst3d_hbm, h_hbm, off, w_hbm):
        pltpu.sync_copy(dst3d_hbm.at[s], dst2d)

        def chunk(j, carry):
            pltpu.sync_copy(src_hbm.at[s, j], srcb)
            d1 = pltpu.async_copy(h_hbm.at[c].at[srcb], rows, sem1)
            d2 = pltpu.async_copy(w_hbm.at[c, s, j], w8, sem2)
            d3 = pltpu.async_copy(den_s.at[dst2d.at[j]], den8, sem3)
            d1.wait()
            d2.wait()
            d3.wait()

            def edge(e2, carry2):
                erow = i16(0) + e2
                wrow = plsc.load_gather(w8, [erow, lane8])
                drow = plsc.load_gather(den8, [erow, lane8])
                arow = wrow / (drow + 1e-16)
                plsc.store_scatter(alpha8, [erow, lane8], arow)
                ab = [plsc.load_gather(alpha8, [erow, cv + off]) for cv in cols]
                for jb in range(FH // 16):
                    hh = (jb * 16) // C
                    rows[e2, pl.ds(jb * 16, 16)] = rows[e2, pl.ds(jb * 16, 16)] * ab[hh]
                return carry2

            lax.fori_loop(0, B, edge, 0, unroll=2)
            pltpu.sync_copy(rows, out_s.at[dst2d.at[j]], add=True)
            return carry

        lax.fori_loop(0, NCH, chunk, 0)

    for t in range(3):
        pltpu.sync_copy(ab1.at[t].at[pl.ds(base, NZ)], ab1_s.at[pl.ds(base, NZ)])
        pltpu.sync_copy(ab2.at[t].at[pl.ds(base, NZ)], ab2_s.at[pl.ds(base, NZ)])
        for k in range(NZ // NZC):
            pltpu.sync_copy(zbuf8, den_s.at[pl.ds(base + k * NZC, NZC)])
        lax.fori_loop(0, B, zr, 0)
        for k in range(NZ // B):
            pltpu.sync_copy(rows, out_s.at[pl.ds(base + k * B, B)])
        pltpu.sync_copy(rows.at[pl.ds(0, NZ - (NZ // B) * B)],
                        out_s.at[pl.ds(base + (NZ // B) * B, NZ - (NZ // B) * B)])
        plsc.subcore_barrier()
        pass_a(src1.at[t], dst1.at[t], ab1_s, 0, w1)
        pass_a(src2.at[t], dst2.at[t], ab2_s, H, w2)
        plsc.subcore_barrier()
        pass_b(src1.at[t], dst1.at[t], h1.at[t], 0, w1)
        pass_b(src2.at[t], dst2.at[t], h2.at[t], H, w2)
        plsc.subcore_barrier()
        pltpu.sync_copy(out_s.at[pl.ds(base, NZ)], out.at[t, c].at[pl.ds(base, NZ)])
        plsc.subcore_barrier()


def _mm_body(act, x_ref, w_ref, o_ref):
    x = x_ref[...]
    if act:
        x = jnp.where(x > 0.0, x, jnp.exp(x) - 1.0)
    o_ref[...] = jnp.dot(x, w_ref[...], preferred_element_type=jnp.float32)


def _mm(x, w, act=False):
    m, k = x.shape
    kc = w.shape[1]
    return pl.pallas_call(
        functools.partial(_mm_body, act),
        out_shape=jax.ShapeDtypeStruct((m, kc), jnp.float32),
        grid=(m // 1000,),
        in_specs=[pl.BlockSpec((1000, k), lambda i: (i, 0)),
                  pl.BlockSpec((k, kc), lambda i: (0, 0))],
        out_specs=pl.BlockSpec((1000, kc), lambda i: (i, 0)),
    )(x, w)


def _normalize_body(x_ref, o_ref):
    x = x_ref[...]
    n = jnp.sqrt(jnp.sum(x * x, axis=1, keepdims=True))
    o_ref[...] = x / jnp.maximum(n, 1e-12)


def _normalize(v):
    return pl.pallas_call(
        _normalize_body,
        out_shape=jax.ShapeDtypeStruct(v.shape, v.dtype),
        grid=(v.shape[0] // 1000,),
        in_specs=[pl.BlockSpec((1000, v.shape[1]), lambda i: (i, 0))],
        out_specs=pl.BlockSpec((1000, v.shape[1]), lambda i: (i, 0)),
    )(v)


@functools.lru_cache(maxsize=None)
def _build_edge_kernel(H, F):
    FH = F // NC
    mesh = plsc.VectorSubcoreMesh(core_axis_name="c", subcore_axis_name="s",
                                  num_cores=NC, num_subcores=NS)
    S = jax.ShapeDtypeStruct
    out_type = (S((3, NC, N, FH), jnp.float32),
                S((NC, NS, NCH, B, 8), jnp.float32),
                S((NC, NS, NCH, B, 8), jnp.float32))
    scratch = [
        pltpu.VMEM((B,), jnp.int32),
        pltpu.VMEM((NCH, B), jnp.int32),
        pltpu.VMEM((B, 8), jnp.float32),
        pltpu.VMEM((B, 8), jnp.float32),
        pltpu.VMEM((B, 8), jnp.float32),
        pltpu.VMEM((B, 8), jnp.float32),
        pltpu.VMEM((B, 8), jnp.float32),
        pltpu.VMEM((B, FH), jnp.float32),
        pltpu.VMEM((NZC, 8), jnp.float32),
        pltpu.SemaphoreType.DMA,
        pltpu.SemaphoreType.DMA,
        pltpu.SemaphoreType.DMA,
        pltpu.VMEM_SHARED((N, 8), jnp.float32),
        pltpu.VMEM_SHARED((N, 8), jnp.float32),
        pltpu.VMEM_SHARED((N, 8), jnp.float32),
        pltpu.VMEM_SHARED((N, FH), jnp.float32),
    ]
    return pl.kernel(
        functools.partial(_edge_body, H, F),
        out_type=out_type,
        mesh=mesh,
        scratch_types=scratch,
        compiler_params=pltpu.CompilerParams(
            needs_layout_passes=False, use_tc_tiling_on_sc=False),
    )


def _layer(xd, eid, params, layer, H, F, act):
    FH = F // NC
    us, ud = {}, {}
    for rel, st, dt in _RELS:
        p = params[layer + "_" + rel]
        W = p["W"]
        din = W.shape[0]
        us[rel] = (W.reshape(din, H, C) * p["as"]).sum(-1)
        ud[rel] = (W.reshape(din, H, C) * p["ad"]).sum(-1)
    types = ("circRNA", "miRNA", "disease")
    P = {}
    cols = {}
    for tau in types:
        srcrels = [r for r, st, dt in _RELS if st == tau]
        dstrels = [r for r, st, dt in _RELS if dt == tau]
        p1, p2 = (params[layer + "_" + r] for r in srcrels)
        wcat = jnp.concatenate(
            [p1["W"], p2["W"], us[srcrels[0]], us[srcrels[1]],
             ud[dstrels[0]], ud[dstrels[1]]], axis=1)
        k = wcat.shape[1]
        kp = ((k + 127) // 128) * 128
        wcat = jnp.pad(wcat, ((0, 0), (0, kp - k)))
        P[tau] = _mm(xd[tau], wcat, act)
        cols[tau] = {srcrels[0]: (0, 2 * F), srcrels[1]: (F, 2 * F + H),
                     dstrels[0]: 2 * F + 2 * H, dstrels[1]: 2 * F + 3 * H}
    hs, abs_ = {}, {}
    for rel, st, dt in _RELS:
        h0, a0 = cols[st][rel]
        h = P[st][:, h0:h0 + F]
        a_s = P[st][:, a0:a0 + H]
        dcol = cols[dt][rel]
        a_d = P[dt][:, dcol:dcol + H]
        hs[rel] = jnp.stack([h[:, i * FH:(i + 1) * FH] for i in range(NC)])
        ab = jnp.zeros((N, 8), jnp.float32)
        ab = ab.at[:, 0:H].set(a_s)
        ab = ab.at[:, 4:4 + H].set(a_d)
        abs_[rel] = ab
    ek = _build_edge_kernel(H, F)
    dts = list(_GROUPS)
    pairs = [_GROUPS[dt] for dt in dts]
    xs = (
        jnp.stack([eid[r1][0].reshape(NS, NCH, B) for r1, _ in pairs]),
        jnp.stack([eid[r1][1].reshape(NS, NCH, B) for r1, _ in pairs]),
        jnp.stack([abs_[r1] for r1, _ in pairs]),
        jnp.stack([hs[r1] for r1, _ in pairs]),
        jnp.stack([eid[r2][0].reshape(NS, NCH, B) for _, r2 in pairs]),
        jnp.stack([eid[r2][1].reshape(NS, NCH, B) for _, r2 in pairs]),
        jnp.stack([abs_[r2] for _, r2 in pairs]),
        jnp.stack([hs[r2] for _, r2 in pairs]),
    )
    aggs, _, _ = ek(*xs)
    out = {}
    for i, dt in enumerate(dts):
        r1, r2 = _GROUPS[dt]
        b = 0.5 * (params[layer + "_" + r1]["b"] + params[layer + "_" + r2]["b"])
        out[dt] = 0.5 * jnp.concatenate([aggs[i, cc] for cc in range(NC)], axis=1) + b
    return out


def kernel(x_circRNA, x_miRNA, x_disease, ei_cm, ei_md, ei_cd, ei_mc, ei_dm, ei_dc, params):
    xd = {"circRNA": x_circRNA, "miRNA": x_miRNA, "disease": x_disease}
    eid = {"cm": ei_cm, "md": ei_md, "cd": ei_cd, "mc": ei_mc, "dm": ei_dm, "dc": ei_dc}
    eid = {k: v.astype(jnp.int32) for k, v in eid.items()}
    x1 = _layer(xd, eid, params, "l1", 4, 256, act=False)
    x2 = _layer(x1, eid, params, "l2", 1, 64, act=True)
    v = jnp.concatenate([x2[t] for t in ("circRNA", "miRNA", "disease")], axis=0)
    return _normalize(v)


# speedup vs baseline: 1.1609x; 1.1609x over previous
import functools

import jax
import jax.numpy as jnp
from jax import lax
from jax.experimental import pallas as pl
from jax.experimental.pallas import tpu as pltpu
from jax.experimental.pallas import tpu_sc as plsc

N = 10000
E = 160000
C = 64
NC, NS = 2, 16
ET = E // NS
B = 80
NCH = ET // B
NZ = N // NS
NZC = 125

_RELS = [("cm", "circRNA", "miRNA"), ("md", "miRNA", "disease"),
         ("cd", "circRNA", "disease"), ("mc", "miRNA", "circRNA"),
         ("dm", "disease", "miRNA"), ("dc", "disease", "circRNA")]
_GROUPS = {"circRNA": ("mc", "dc"), "miRNA": ("cm", "dm"), "disease": ("md", "cd")}


def _edge_body(H, F,
               src1, dst1, ab1, h1,
               src2, dst2, ab2, h2,
               out, w1, w2,
               srcall, dst2d, w8, den8, alpha8, asrows, adrows, rows, zbuf8,
               sem1, sem2, sem3, ab1_s, ab2_s, den_s, out_s):
    FH = F // NC
    c = lax.axis_index("c")
    s = lax.axis_index("s")
    lane = lax.broadcasted_iota(jnp.int32, (16,), 0)
    lane8 = jnp.bitwise_and(lane, 7)
    lane4p = jnp.bitwise_and(lane, 3) + 4
    mh = lane < H
    z16 = jnp.zeros((16,), jnp.float32)
    base = s * NZ

    def i16(v):
        return jnp.full((16,), v, jnp.int32)

    def zw(i, carry):
        plsc.store_scatter(w8, [i16(i), lane8], z16)
        return carry

    def zr(i, carry):
        for jb in range(FH // 16):
            rows[i, pl.ds(jb * 16, 16)] = z16
        return carry

    def zb(i, carry):
        plsc.store_scatter(zbuf8, [i16(i), lane8], z16)
        return carry

    lax.fori_loop(0, NZC, zb, 0)

    def pass_a(src_hbm, dst3d_hbm, ab_s, off, w_hbm):
        lax.fori_loop(0, B, zw, 0)
        pltpu.sync_copy(dst3d_hbm.at[s], dst2d)
        pltpu.sync_copy(src_hbm.at[s], srcall)

        def chunk(j, carry):
            d1 = pltpu.async_copy(ab_s.at[srcall.at[j]], asrows, sem1)
            d2 = pltpu.async_copy(ab_s.at[dst2d.at[j]], adrows, sem2)
            d1.wait()
            d2.wait()

            def edge(e2, carry2):
                erow = i16(0) + e2
                av = plsc.load_gather(asrows, [erow, lane8])
                dv = plsc.load_gather(adrows, [erow, lane4p])
                x = av + dv
                e = jnp.maximum(x, 0.2 * x)
                w = jnp.exp(e)
                plsc.store_scatter(w8, [erow, lane + off], w, mask=mh)
                return carry2

            lax.fori_loop(0, B, edge, 0, unroll=2)
            pltpu.sync_copy(w8, den_s.at[dst2d.at[j]], add=True)
            pltpu.sync_copy(w8, w_hbm.at[c, s, j])
            return carry

        lax.fori_loop(0, NCH, chunk, 0)

    head_base = (c * FH) // C
    nh = max(1, FH // C)
    cols = [i16(0) + (head_base + hh) for hh in range(nh)]

    def pass_b(src_hbm, dst3d_hbm, h_hbm, off, w_hbm):
        pltpu.sync_copy(dst3d_hbm.at[s], dst2d)
        pltpu.sync_copy(src_hbm.at[s], srcall)

        def chunk(j, carry):
            d1 = pltpu.async_copy(h_hbm.at[c].at[srcall.at[j]], rows, sem1)
            d2 = pltpu.async_copy(w_hbm.at[c, s, j], w8, sem2)
            d3 = pltpu.async_copy(den_s.at[dst2d.at[j]], den8, sem3)
            d1.wait()
            d2.wait()
            d3.wait()

            def edge(e2, carry2):
                erow = i16(0) + e2
                wrow = plsc.load_gather(w8, [erow, lane8])
                drow = plsc.load_gather(den8, [erow, lane8])
                arow = wrow / (drow + 1e-16)
                plsc.store_scatter(alpha8, [erow, lane8], arow)
                ab = [plsc.load_gather(alpha8, [erow, cv + off]) for cv in cols]
                for jb in range(FH // 16):
                    hh = (jb * 16) // C
                    rows[e2, pl.ds(jb * 16, 16)] = rows[e2, pl.ds(jb * 16, 16)] * ab[hh]
                return carry2

            lax.fori_loop(0, B, edge, 0, unroll=2)
            pltpu.sync_copy(rows, out_s.at[dst2d.at[j]], add=True)
            return carry

        lax.fori_loop(0, NCH, chunk, 0)

    for t in range(3):
        pltpu.sync_copy(ab1.at[t].at[pl.ds(base, NZ)], ab1_s.at[pl.ds(base, NZ)])
        pltpu.sync_copy(ab2.at[t].at[pl.ds(base, NZ)], ab2_s.at[pl.ds(base, NZ)])
        for k in range(NZ // NZC):
            pltpu.sync_copy(zbuf8, den_s.at[pl.ds(base + k * NZC, NZC)])
        lax.fori_loop(0, B, zr, 0)
        for k in range(NZ // B):
            pltpu.sync_copy(rows, out_s.at[pl.ds(base + k * B, B)])
        pltpu.sync_copy(rows.at[pl.ds(0, NZ - (NZ // B) * B)],
                        out_s.at[pl.ds(base + (NZ // B) * B, NZ - (NZ // B) * B)])
        plsc.subcore_barrier()
        pass_a(src1.at[t], dst1.at[t], ab1_s, 0, w1)
        pass_a(src2.at[t], dst2.at[t], ab2_s, H, w2)
        plsc.subcore_barrier()
        pass_b(src1.at[t], dst1.at[t], h1.at[t], 0, w1)
        pass_b(src2.at[t], dst2.at[t], h2.at[t], H, w2)
        plsc.subcore_barrier()
        pltpu.sync_copy(out_s.at[pl.ds(base, NZ)], out.at[t, c].at[pl.ds(base, NZ)])
        plsc.subcore_barrier()


def _mm_body(act, x_ref, w_ref, o_ref):
    x = x_ref[...]
    if act:
        x = jnp.where(x > 0.0, x, jnp.exp(x) - 1.0)
    o_ref[...] = jnp.dot(x, w_ref[...], preferred_element_type=jnp.float32)


def _mm(x, w, act=False):
    m, k = x.shape
    kc = w.shape[1]
    return pl.pallas_call(
        functools.partial(_mm_body, act),
        out_shape=jax.ShapeDtypeStruct((m, kc), jnp.float32),
        grid=(m // 1000,),
        in_specs=[pl.BlockSpec((1000, k), lambda i: (i, 0)),
                  pl.BlockSpec((k, kc), lambda i: (0, 0))],
        out_specs=pl.BlockSpec((1000, kc), lambda i: (i, 0)),
    )(x, w)


def _normalize_body(x_ref, o_ref):
    x = x_ref[...]
    n = jnp.sqrt(jnp.sum(x * x, axis=1, keepdims=True))
    o_ref[...] = x / jnp.maximum(n, 1e-12)


def _normalize(v):
    return pl.pallas_call(
        _normalize_body,
        out_shape=jax.ShapeDtypeStruct(v.shape, v.dtype),
        grid=(v.shape[0] // 1000,),
        in_specs=[pl.BlockSpec((1000, v.shape[1]), lambda i: (i, 0))],
        out_specs=pl.BlockSpec((1000, v.shape[1]), lambda i: (i, 0)),
    )(v)


@functools.lru_cache(maxsize=None)
def _build_edge_kernel(H, F):
    FH = F // NC
    mesh = plsc.VectorSubcoreMesh(core_axis_name="c", subcore_axis_name="s",
                                  num_cores=NC, num_subcores=NS)
    S = jax.ShapeDtypeStruct
    out_type = (S((3, NC, N, FH), jnp.float32),
                S((NC, NS, NCH, B, 8), jnp.float32),
                S((NC, NS, NCH, B, 8), jnp.float32))
    scratch = [
        pltpu.VMEM((NCH, B), jnp.int32),
        pltpu.VMEM((NCH, B), jnp.int32),
        pltpu.VMEM((B, 8), jnp.float32),
        pltpu.VMEM((B, 8), jnp.float32),
        pltpu.VMEM((B, 8), jnp.float32),
        pltpu.VMEM((B, 8), jnp.float32),
        pltpu.VMEM((B, 8), jnp.float32),
        pltpu.VMEM((B, FH), jnp.float32),
        pltpu.VMEM((NZC, 8), jnp.float32),
        pltpu.SemaphoreType.DMA,
        pltpu.SemaphoreType.DMA,
        pltpu.SemaphoreType.DMA,
        pltpu.VMEM_SHARED((N, 8), jnp.float32),
        pltpu.VMEM_SHARED((N, 8), jnp.float32),
        pltpu.VMEM_SHARED((N, 8), jnp.float32),
        pltpu.VMEM_SHARED((N, FH), jnp.float32),
    ]
    return pl.kernel(
        functools.partial(_edge_body, H, F),
        out_type=out_type,
        mesh=mesh,
        scratch_types=scratch,
        compiler_params=pltpu.CompilerParams(
            needs_layout_passes=False, use_tc_tiling_on_sc=False),
    )


def _layer(xd, eid, params, layer, H, F, act):
    FH = F // NC
    us, ud = {}, {}
    for rel, st, dt in _RELS:
        p = params[layer + "_" + rel]
        W = p["W"]
        din = W.shape[0]
        us[rel] = (W.reshape(din, H, C) * p["as"]).sum(-1)
        ud[rel] = (W.reshape(din, H, C) * p["ad"]).sum(-1)
    types = ("circRNA", "miRNA", "disease")
    P = {}
    cols = {}
    for tau in types:
        srcrels = [r for r, st, dt in _RELS if st == tau]
        dstrels = [r for r, st, dt in _RELS if dt == tau]
        p1, p2 = (params[layer + "_" + r] for r in srcrels)
        wcat = jnp.concatenate(
            [p1["W"], p2["W"], us[srcrels[0]], us[srcrels[1]],
             ud[dstrels[0]], ud[dstrels[1]]], axis=1)
        k = wcat.shape[1]
        kp = ((k + 127) // 128) * 128
        wcat = jnp.pad(wcat, ((0, 0), (0, kp - k)))
        P[tau] = _mm(xd[tau], wcat, act)
        cols[tau] = {srcrels[0]: (0, 2 * F), srcrels[1]: (F, 2 * F + H),
                     dstrels[0]: 2 * F + 2 * H, dstrels[1]: 2 * F + 3 * H}
    hs, abs_ = {}, {}
    for rel, st, dt in _RELS:
        h0, a0 = cols[st][rel]
        h = P[st][:, h0:h0 + F]
        a_s = P[st][:, a0:a0 + H]
        dcol = cols[dt][rel]
        a_d = P[dt][:, dcol:dcol + H]
        hs[rel] = jnp.stack([h[:, i * FH:(i + 1) * FH] for i in range(NC)])
        ab = jnp.zeros((N, 8), jnp.float32)
        ab = ab.at[:, 0:H].set(a_s)
        ab = ab.at[:, 4:4 + H].set(a_d)
        abs_[rel] = ab
    ek = _build_edge_kernel(H, F)
    dts = list(_GROUPS)
    pairs = [_GROUPS[dt] for dt in dts]
    xs = (
        jnp.stack([eid[r1][0].reshape(NS, NCH, B) for r1, _ in pairs]),
        jnp.stack([eid[r1][1].reshape(NS, NCH, B) for r1, _ in pairs]),
        jnp.stack([abs_[r1] for r1, _ in pairs]),
        jnp.stack([hs[r1] for r1, _ in pairs]),
        jnp.stack([eid[r2][0].reshape(NS, NCH, B) for _, r2 in pairs]),
        jnp.stack([eid[r2][1].reshape(NS, NCH, B) for _, r2 in pairs]),
        jnp.stack([abs_[r2] for _, r2 in pairs]),
        jnp.stack([hs[r2] for _, r2 in pairs]),
    )
    aggs, _, _ = ek(*xs)
    out = {}
    for i, dt in enumerate(dts):
        r1, r2 = _GROUPS[dt]
        b = 0.5 * (params[layer + "_" + r1]["b"] + params[layer + "_" + r2]["b"])
        out[dt] = 0.5 * jnp.concatenate([aggs[i, cc] for cc in range(NC)], axis=1) + b
    return out


def kernel(x_circRNA, x_miRNA, x_disease, ei_cm, ei_md, ei_cd, ei_mc, ei_dm, ei_dc, params):
    xd = {"circRNA": x_circRNA, "miRNA": x_miRNA, "disease": x_disease}
    eid = {"cm": ei_cm, "md": ei_md, "cd": ei_cd, "mc": ei_mc, "dm": ei_dm, "dc": ei_dc}
    eid = {k: v.astype(jnp.int32) for k, v in eid.items()}
    x1 = _layer(xd, eid, params, "l1", 4, 256, act=False)
    x2 = _layer(x1, eid, params, "l2", 1, 64, act=True)
    v = jnp.concatenate([x2[t] for t in ("circRNA", "miRNA", "disease")], axis=0)
    return _normalize(v)
